# baseline (speedup 1.0000x reference)
import functools

import jax
import jax.numpy as jnp
from jax import lax
from jax.experimental import pallas as pl
from jax.experimental.pallas import tpu as pltpu
from jax.experimental.pallas import tpu_sc as plsc

VOCAB_SIZE = 1000
N_ROWS = 4096 * 26
TOTAL_WORDS = N_ROWS * VOCAB_SIZE
NUM_WORKERS = 32
ROWS_PER_WORKER = N_ROWS // NUM_WORKERS
SCATTER_BATCH = 128
NUM_BATCHES = ROWS_PER_WORKER // SCATTER_BATCH

_ZR, _ZC = 832, 128000
_ZBLK = 8


def _zero_body(out_ref):
    out_ref[...] = jnp.zeros((_ZBLK, _ZC), jnp.float32)


_zero_canvas = pl.pallas_call(
    _zero_body,
    out_shape=jax.ShapeDtypeStruct((_ZR, _ZC), jnp.float32),
    grid=(_ZR // _ZBLK,),
    out_specs=pl.BlockSpec((_ZBLK, _ZC), lambda i: (i, 0)),
)

_mesh = plsc.VectorSubcoreMesh(core_axis_name="c", subcore_axis_name="s")


@functools.partial(
    pl.kernel,
    out_type=(),
    mesh=_mesh,
    scratch_types=[
        pltpu.VMEM((ROWS_PER_WORKER,), jnp.int32),
        pltpu.VMEM((NUM_BATCHES, SCATTER_BATCH), jnp.int32),
        pltpu.VMEM((SCATTER_BATCH,), jnp.float32),
        pltpu.SemaphoreType.DMA,
    ],
)
def _scatter_ones(idx_hbm, canvas_ref, idx_v, pos_v, ones_v, sem):
    wid = lax.axis_index("c") * 16 + lax.axis_index("s")
    row0 = wid * ROWS_PER_WORKER

    pltpu.sync_copy(idx_hbm.at[pl.ds(row0, ROWS_PER_WORKER)], idx_v)

    lane = lax.iota(jnp.int32, 16)
    for c in range(SCATTER_BATCH // 16):
        ones_v[pl.ds(c * 16, 16)] = jnp.full((16,), 1.0, jnp.float32)

    def compute_pos(j, _):
        for c in range(SCATTER_BATCH // 16):
            r = j * SCATTER_BATCH + c * 16
            v = idx_v[pl.ds(r, 16)]
            pos_v[j, pl.ds(c * 16, 16)] = (row0 + r + lane) * VOCAB_SIZE + v
        return 0

    lax.fori_loop(0, NUM_BATCHES, compute_pos, 0)

    def fire(j, _):
        pltpu.async_copy(ones_v, canvas_ref.at[pos_v.at[j]], sem)
        return 0

    lax.fori_loop(0, NUM_BATCHES, fire, 0)

    def drain(j, _):
        pltpu.make_async_copy(ones_v, canvas_ref.at[pos_v.at[j]], sem).wait()
        return 0

    lax.fori_loop(0, NUM_BATCHES, drain, 0)


def kernel(x):
    idx = x.astype(jnp.int32).reshape(-1)
    canvas = _zero_canvas().reshape(TOTAL_WORDS)
    ref = jax.new_ref(canvas)
    _scatter_ones(idx, ref)
    return ref[...].reshape(4096, 26, VOCAB_SIZE)


# speedup vs baseline: 1.0890x; 1.0890x over previous
import functools

import jax
import jax.numpy as jnp
from jax import lax
from jax.experimental import pallas as pl
from jax.experimental.pallas import tpu as pltpu
from jax.experimental.pallas import tpu_sc as plsc

VOCAB_SIZE = 1000
N_ROWS = 4096 * 26
TOTAL_WORDS = N_ROWS * VOCAB_SIZE
NUM_WORKERS = 32
ROWS_PER_WORKER = N_ROWS // NUM_WORKERS
SCATTER_BATCH = 128
NUM_BATCHES = ROWS_PER_WORKER // SCATTER_BATCH

_ZR, _ZC = 832, 128000
_ZBLK = 8


def _zero_body(out_ref):
    out_ref[...] = jnp.zeros((_ZBLK, _ZC), jnp.float32)


_zero_canvas = pl.pallas_call(
    _zero_body,
    out_shape=jax.ShapeDtypeStruct((_ZR, _ZC), jnp.float32),
    grid=(_ZR // _ZBLK,),
    out_specs=pl.BlockSpec((_ZBLK, _ZC), lambda i: (i, 0)),
)

_mesh = plsc.VectorSubcoreMesh(core_axis_name="c", subcore_axis_name="s")


@functools.partial(
    pl.kernel,
    out_type=(),
    mesh=_mesh,
    scratch_types=[
        pltpu.VMEM((ROWS_PER_WORKER,), jnp.int32),
        pltpu.VMEM((NUM_BATCHES, SCATTER_BATCH), jnp.int32),
        pltpu.VMEM((SCATTER_BATCH,), jnp.float32),
        pltpu.SemaphoreType.DMA,
    ],
)
def _scatter_ones(idx_hbm, canvas_ref, idx_v, pos_v, ones_v, sem):
    wid = lax.axis_index("c") * 16 + lax.axis_index("s")
    row0 = wid * ROWS_PER_WORKER

    pltpu.sync_copy(idx_hbm.at[pl.ds(row0, ROWS_PER_WORKER)], idx_v)

    lane = lax.iota(jnp.int32, 16)
    for c in range(SCATTER_BATCH // 16):
        ones_v[pl.ds(c * 16, 16)] = jnp.full((16,), 1.0, jnp.float32)

    def compute_pos(j, _):
        for c in range(SCATTER_BATCH // 16):
            r = j * SCATTER_BATCH + c * 16
            v = idx_v[pl.ds(r, 16)]
            pos_v[j, pl.ds(c * 16, 16)] = (row0 + r + lane) * VOCAB_SIZE + v
        return 0

    lax.fori_loop(0, NUM_BATCHES, compute_pos, 0)

    def fire(j, _):
        pltpu.async_copy(ones_v, canvas_ref.at[pos_v.at[j]], sem)
        return 0

    lax.fori_loop(0, NUM_BATCHES, fire, 0)

    def drain(j, _):
        pltpu.make_async_copy(ones_v, canvas_ref.at[pos_v.at[j]], sem).wait()
        return 0

    lax.fori_loop(0, NUM_BATCHES, drain, 0)


def kernel(x):
    idx = x.astype(jnp.int32).reshape(-1)
    canvas = _zero_canvas().reshape(TOTAL_WORDS)
    return canvas.reshape(4096, 26, VOCAB_SIZE)


# speedup vs baseline: 2.7007x; 2.4800x over previous
import jax
import jax.numpy as jnp
from jax.experimental import pallas as pl
from jax.experimental.pallas import tpu as pltpu

VOCAB_SIZE = 1000
BLK = 64


def _body(x_ref, out_ref):
    idx = x_ref[...]
    iota = jax.lax.broadcasted_iota(jnp.int32, (BLK, 26, VOCAB_SIZE), 2)
    out_ref[...] = (iota == idx[:, :, None]).astype(jnp.float32)


_one_hot_tc = pl.pallas_call(
    _body,
    out_shape=jax.ShapeDtypeStruct((4096, 26, VOCAB_SIZE), jnp.float32),
    grid=(4096 // BLK,),
    in_specs=[pl.BlockSpec((BLK, 26), lambda i: (i, 0))],
    out_specs=pl.BlockSpec((BLK, 26, VOCAB_SIZE), lambda i: (i, 0, 0)),
)


def kernel(x):
    return _one_hot_tc(x.astype(jnp.int32))


# speedup vs baseline: 12.5555x; 4.6490x over previous
import jax
import jax.numpy as jnp
from jax import lax
from jax.experimental import pallas as pl

VOCAB_SIZE = 1000


def _body(xt_ref, out_ref):
    idx = xt_ref[...]
    kio = lax.broadcasted_iota(jnp.int32, (1, VOCAB_SIZE, 4096), 1)
    out_ref[...] = (kio == idx).astype(jnp.float32)


_one_hot_t = pl.pallas_call(
    _body,
    out_shape=jax.ShapeDtypeStruct((26, VOCAB_SIZE, 4096), jnp.float32),
    grid=(26,),
    in_specs=[pl.BlockSpec((1, 1, 4096), lambda j: (j, 0, 0))],
    out_specs=pl.BlockSpec((1, VOCAB_SIZE, 4096), lambda j: (j, 0, 0)),
)


def kernel(x):
    xt = x.astype(jnp.int32).T.reshape(26, 1, 4096)
    y = _one_hot_t(xt)
    return jnp.transpose(y, (2, 0, 1))


# speedup vs baseline: 12.7913x; 1.0188x over previous
import jax
import jax.numpy as jnp
from jax import lax
from jax.experimental import pallas as pl

VOCAB_SIZE = 1000


def _body(xt_ref, out_ref):
    idx = xt_ref[...]
    kio = lax.broadcasted_iota(jnp.int32, (1, VOCAB_SIZE, 2048), 1)
    out_ref[...] = (kio == idx).astype(jnp.float32)


_one_hot_t = pl.pallas_call(
    _body,
    out_shape=jax.ShapeDtypeStruct((26, VOCAB_SIZE, 4096), jnp.float32),
    grid=(26, 2),
    in_specs=[pl.BlockSpec((1, 1, 2048), lambda j, i: (j, 0, i))],
    out_specs=pl.BlockSpec((1, VOCAB_SIZE, 2048), lambda j, i: (j, 0, i)),
)


def kernel(x):
    xt = x.astype(jnp.int32).T.reshape(26, 1, 4096)
    y = _one_hot_t(xt)
    return jnp.transpose(y, (2, 0, 1))


# speedup vs baseline: 12.9561x; 1.0129x over previous
import jax
import jax.numpy as jnp
from jax import lax
from jax.experimental import pallas as pl

VOCAB_SIZE = 1000


def _body(xt_ref, out_ref):
    idx = xt_ref[...]
    kio = lax.broadcasted_iota(jnp.int32, (1, VOCAB_SIZE, 1024), 1)
    out_ref[...] = (kio == idx).astype(jnp.float32)


_one_hot_t = pl.pallas_call(
    _body,
    out_shape=jax.ShapeDtypeStruct((26, VOCAB_SIZE, 4096), jnp.float32),
    grid=(26, 4),
    in_specs=[pl.BlockSpec((1, 1, 1024), lambda j, i: (j, 0, i))],
    out_specs=pl.BlockSpec((1, VOCAB_SIZE, 1024), lambda j, i: (j, 0, i)),
)


def kernel(x):
    xt = x.astype(jnp.int32).T.reshape(26, 1, 4096)
    y = _one_hot_t(xt)
    return jnp.transpose(y, (2, 0, 1))
